# baseline (speedup 1.0000x reference)
import functools

import jax
import jax.numpy as jnp
from jax import lax
from jax.experimental import pallas as pl
from jax.experimental.pallas import tpu as pltpu
from jax.experimental.pallas import tpu_sc as plsc

N = 10000
E = 320000
D = 128
NPAD = 10240
NW = 32
EPW = 10240
EPAD = EPW * NW
CH = 128
NCH = EPW // CH
RPS = NPAD // 16
NQ = RPS // CH

_f32 = jnp.float32

_mesh = plsc.VectorSubcoreMesh(core_axis_name="c", subcore_axis_name="s")


CG = 40
IB = 32
NBT = 16
SEG_B0 = 4
NQ2 = RPS // CG


@functools.partial(
    pl.kernel,
    out_type=[
        jax.ShapeDtypeStruct((2, NPAD, D), _f32),
    ],
    scratch_types=[
        pltpu.VMEM((IB, CG), jnp.int32),
        pltpu.VMEM((IB, CG), jnp.int32),
        pltpu.VMEM((4, CG, D), _f32),
        pltpu.VMEM_SHARED((NPAD, D), _f32),
        pltpu.SemaphoreType.DMA,
        pltpu.SemaphoreType.DMA,
    ],
    mesh=_mesh,
)
def _sc_segmean(table, src2, dst2, zrows, parts,
                sidx2, didx2, rows2,
                acc, gsem, ssem):
    cid = lax.axis_index("c")
    sid = lax.axis_index("s")

    pltpu.sync_copy(zrows, acc.at[pl.ds(sid * RPS, RPS), :])

    plsc.subcore_barrier()

    def _fire_gather(i, b):
        pltpu.async_copy(table.at[sidx2.at[i]], rows2.at[b], gsem)

    def _wait_gather(i, b):
        pltpu.make_async_copy(table.at[sidx2.at[i]], rows2.at[b], gsem).wait()

    def _fire_scat(i, b):
        pltpu.async_copy(rows2.at[b], acc.at[didx2.at[i]], ssem, add=True)

    def _wait_scat(i, b):
        pltpu.make_async_copy(rows2.at[b], acc.at[didx2.at[i]], ssem).wait()

    nb = jnp.where(cid == 0, SEG_B0, NBT - SEG_B0)
    b0 = jnp.where(cid == 0, sid * SEG_B0,
                   16 * SEG_B0 + sid * (NBT - SEG_B0))

    def _batch(bi, carry):
        ch0 = pl.multiple_of((b0 + bi) * IB, 8)
        pltpu.sync_copy(src2.at[pl.ds(ch0, IB)], sidx2)
        pltpu.sync_copy(dst2.at[pl.ds(ch0, IB)], didx2)
        for b in range(4):
            _fire_gather(b, b)

        def _inner(o, carry2):
            for b in range(4):
                i = o * 4 + b
                _wait_gather(i, b)
                _fire_scat(i, b)
            for b in range(4):
                i = o * 4 + b
                _wait_scat(i, b)
                _fire_gather(i + 4, b)
            return carry2
        lax.fori_loop(0, IB // 4 - 1, _inner, 0)

        for b in range(4):
            i = IB - 4 + b
            _wait_gather(i, b)
            _fire_scat(i, b)
        for b in range(4):
            _wait_scat(IB - 4 + b, b)
        return carry
    lax.fori_loop(0, nb, _batch, 0)

    plsc.subcore_barrier()

    pltpu.sync_copy(acc.at[pl.ds(sid * RPS, RPS), :],
                    parts.at[cid, pl.ds(sid * RPS, RPS), :])


@functools.partial(
    pl.kernel,
    out_type=[
        jax.ShapeDtypeStruct((2, NPAD, D), _f32),
    ],
    scratch_types=[
        pltpu.VMEM((NCH, CH), jnp.int32),
        pltpu.VMEM((CH, D), _f32),
        pltpu.VMEM_SHARED((NPAD, D), _f32),
        pltpu.SemaphoreType.DMA,
    ],
    mesh=_mesh,
)
def _sc_cnt(dst3, zrows, cnt_parts, didx2, onesb, cacc, ssem):
    cid = lax.axis_index("c")
    sid = lax.axis_index("s")
    wid = cid * 16 + sid

    pltpu.sync_copy(dst3.at[wid], didx2)
    pltpu.sync_copy(zrows, cacc.at[pl.ds(sid * RPS, RPS), :])

    one16 = jnp.full((16,), 1.0, _f32)

    def _fill(r, carry):
        for j in range(D // 16):
            onesb[r, pl.ds(j * 16, 16)] = one16
        return carry
    lax.fori_loop(0, CH, _fill, 0)
    plsc.subcore_barrier()

    def _fire(i):
        pltpu.async_copy(onesb, cacc.at[didx2.at[i]], ssem, add=True)

    def _drain(i):
        pltpu.make_async_copy(onesb, cacc.at[didx2.at[i]], ssem).wait()

    for b in range(8):
        _fire(b)

    def _outer(o, carry):
        for b in range(8):
            i = o * 8 + b
            _drain(i)
            _fire(i + 8)
        return carry
    lax.fori_loop(0, NCH // 8 - 1, _outer, 0)

    for b in range(8):
        _drain(NCH - 8 + b)

    plsc.subcore_barrier()
    pltpu.sync_copy(cacc.at[pl.ds(sid * RPS, RPS), :],
                    cnt_parts.at[cid, pl.ds(sid * RPS, RPS), :])


CE = 64
RE = 4
EB = 16
NBT_E = 20
EDGE_B0 = 5


@functools.partial(
    pl.kernel,
    out_type=[jax.ShapeDtypeStruct((EPAD * 16,), _f32)],
    scratch_types=[
        pltpu.VMEM((EB, CE), jnp.int32),
        pltpu.VMEM((EB, CE), jnp.int32),
        pltpu.VMEM((RE, CE, D), _f32),
        pltpu.VMEM((RE, CE, D), _f32),
        pltpu.VMEM((D,), _f32),
        pltpu.VMEM((CE * 16,), _f32),
        pltpu.SemaphoreType.DMA,
    ],
    mesh=_mesh,
)
def _sc_edge(A, B, src2, dst2, wp2, tsum,
             sidx2, didx2, rA, rB, wvec, outb, gsem):
    cid = lax.axis_index("c")
    sid = lax.axis_index("s")

    pltpu.sync_copy(wp2, wvec)
    wl = [wvec[pl.ds(j * 16, 16)] for j in range(D // 16)]

    def _fire(i, k):
        pltpu.async_copy(A.at[sidx2.at[i]], rA.at[k], gsem)
        pltpu.async_copy(B.at[didx2.at[i]], rB.at[k], gsem)

    def _wait(i, k):
        pltpu.make_async_copy(A.at[sidx2.at[i]], rA.at[k], gsem).wait()
        pltpu.make_async_copy(B.at[didx2.at[i]], rB.at[k], gsem).wait()

    nb = jnp.where(cid == 0, EDGE_B0, NBT_E - EDGE_B0)
    b0 = jnp.where(cid == 0, sid * EDGE_B0,
                   16 * EDGE_B0 + sid * (NBT_E - EDGE_B0))

    def _batch(bi, carry):
        ch0 = pl.multiple_of((b0 + bi) * EB, 8)
        pltpu.sync_copy(src2.at[pl.ds(ch0, EB)], sidx2)
        pltpu.sync_copy(dst2.at[pl.ds(ch0, EB)], didx2)

        def _compute(i, k):
            def _edge(e, carry2):
                t = jnp.zeros((16,), _f32)
                for j in range(D // 16):
                    va = rA[k, e, pl.ds(j * 16, 16)]
                    vb = rB[k, e, pl.ds(j * 16, 16)]
                    t = t + jnp.maximum(va + vb, 0.0) * wl[j]
                outb[pl.ds(e * 16, 16)] = t
                return carry2
            lax.fori_loop(0, CE, _edge, 0)
            pltpu.sync_copy(outb,
                            tsum.at[pl.ds((ch0 + i) * CE * 16, CE * 16)])

        for k in range(RE):
            _fire(k, k)

        def _outer(o, carry2):
            for k in range(RE):
                i = o * RE + k
                _wait(i, k)
                _compute(i, k)
                _fire(i + RE, k)
            return carry2
        lax.fori_loop(0, EB // RE - 1, _outer, 0)

        for k in range(RE):
            i = EB - RE + k
            _wait(i, k)
            _compute(i, k)
        return carry
    lax.fori_loop(0, nb, _batch, 0)


def _mmT(a, w):
    return lax.dot_general(a, w, (((1,), (1,)), ((), ())),
                           preferred_element_type=_f32)


def _tc_inv_body(cnt_ref, inv_ref):
    cnt = cnt_ref[0, :, 0] + cnt_ref[1, :, 0]
    inv_ref[...] = (1.0 / jnp.maximum(cnt, 1.0))[:, None]


def _tc_pre_body(x_ref, wl_ref, wr_ref, g_ref, r_ref):
    x = x_ref[...]
    g_ref[...] = _mmT(x, wl_ref[...])
    r_ref[...] = _mmT(x, wr_ref[...])


def _tc_mid_body(parts_ref, inv_ref, r_ref, bl_ref, wln_ref, wrn_ref,
                 g_ref, rn_ref):
    h = jnp.maximum((parts_ref[0] + parts_ref[1]) * inv_ref[...]
                    + bl_ref[...] + r_ref[...], 0.0)
    g_ref[...] = _mmT(h, wln_ref[...])
    rn_ref[...] = _mmT(h, wrn_ref[...])


def _tc_fin_body(parts_ref, inv_ref, r_ref, bl_ref, wc_ref, bc_ref,
                 wps_ref, wpd_ref, bp1_ref, ne_ref, sup_ref, a_ref, b_ref):
    ne = (parts_ref[0] + parts_ref[1]) * inv_ref[...] + bl_ref[...] + r_ref[...]
    ne_ref[...] = ne
    sup_ref[...] = _mmT(ne, wc_ref[...]) + bc_ref[...]
    a_ref[...] = _mmT(ne, wps_ref[...]) + bp1_ref[...]
    b_ref[...] = _mmT(ne, wpd_ref[...])


def _tc_edge_fin_body(t_ref, m_ref, out_ref):
    out_ref[...] = lax.dot_general(t_ref[...], m_ref[...],
                                   (((1,), (0,)), ((), ())),
                                   preferred_element_type=_f32)


def _tc(body, out_shape, *args):
    return pl.pallas_call(body, out_shape=out_shape)(*args)


def kernel(x, edge_index, Wl1, bl1, Wr1, Wl2, bl2, Wr2, Wl3, bl3, Wr3,
           Wp1, bp1, Wp2, bp2, Wc, bc):
    src = edge_index[0]
    dst = edge_index[1]
    src_p = jnp.concatenate([src, jnp.zeros((EPAD - E,), jnp.int32)])
    dst_p = jnp.concatenate([dst, jnp.full((EPAD - E,), N + 100, jnp.int32)])
    src2g = src_p.reshape(EPAD // CG, CG)
    dst2g = dst_p.reshape(EPAD // CG, CG)
    src2e = src_p.reshape(EPAD // CE, CE)
    dst2e = dst_p.reshape(EPAD // CE, CE)
    dst3c = dst_p.reshape(NW, NCH, CH)
    x_p = jnp.pad(x, ((0, NPAD - N), (0, 0)))
    zrows = jnp.zeros((RPS, D), _f32)
    Wc_p = jnp.pad(Wc, ((0, D - Wc.shape[0]), (0, 0)))
    bc_p = jnp.pad(bc, (0, D - bc.shape[0])).reshape(1, D)
    Wp1s = Wp1[:, :D]
    Wp1d = Wp1[:, D:]

    sds = jax.ShapeDtypeStruct
    g1, r1 = _tc(_tc_pre_body,
                 [sds((NPAD, D), _f32), sds((NPAD, D), _f32)],
                 x_p, Wl1, Wr1)
    (parts1,) = _sc_segmean(g1, src2g, dst2g, zrows)
    (cnt_parts,) = _sc_cnt(dst3c, zrows)
    inv = _tc(_tc_inv_body, sds((NPAD, 1), _f32), cnt_parts)

    g2, r2 = _tc(_tc_mid_body,
                 [sds((NPAD, D), _f32), sds((NPAD, D), _f32)],
                 parts1, inv, r1, bl1.reshape(1, D), Wl2, Wr2)
    (parts2,) = _sc_segmean(g2, src2g, dst2g, zrows)

    g3, r3 = _tc(_tc_mid_body,
                 [sds((NPAD, D), _f32), sds((NPAD, D), _f32)],
                 parts2, inv, r2, bl2.reshape(1, D), Wl3, Wr3)
    (parts3,) = _sc_segmean(g3, src2g, dst2g, zrows)

    ne, sup, Ab, Bb = _tc(
        _tc_fin_body,
        [sds((NPAD, D), _f32), sds((NPAD, D), _f32),
         sds((NPAD, D), _f32), sds((NPAD, D), _f32)],
        parts3, inv, r3, bl3.reshape(1, D), Wc_p, bc_p,
        Wp1s, Wp1d, bp1.reshape(1, D))

    (tsum,) = _sc_edge(Ab, Bb, src2e, dst2e, Wp2.reshape(D))
    msel = jnp.repeat(jnp.eye(8, dtype=_f32), 16, axis=0)
    flow = _tc(_tc_edge_fin_body, sds((EPAD // 8, 8), _f32),
               tsum.reshape(EPAD // 8, D), msel)

    node_embeddings = ne[:N]
    carbon_flows = flow.reshape(EPAD, 1)[:E] + bp2
    supplier_classes = sup[:N, :4]
    return (node_embeddings, carbon_flows, supplier_classes)


# speedup vs baseline: 1.1918x; 1.1918x over previous
import functools

import jax
import jax.numpy as jnp
from jax import lax
from jax.experimental import pallas as pl
from jax.experimental.pallas import tpu as pltpu
from jax.experimental.pallas import tpu_sc as plsc

N = 10000
E = 320000
D = 128
NPAD = 10240
NW = 32
EPW = 10240
EPAD = EPW * NW
CH = 128
NCH = EPW // CH
RPS = NPAD // 16
NQ = RPS // CH

_f32 = jnp.float32

_mesh = plsc.VectorSubcoreMesh(core_axis_name="c", subcore_axis_name="s")


CG = 40
IB = 32
NBT = 16
SEG_B0 = 12
NQ2 = RPS // CG


@functools.partial(
    pl.kernel,
    out_type=[
        jax.ShapeDtypeStruct((2, NPAD, D), _f32),
    ],
    scratch_types=[
        pltpu.VMEM((IB, CG), jnp.int32),
        pltpu.VMEM((IB, CG), jnp.int32),
        pltpu.VMEM((4, CG, D), _f32),
        pltpu.VMEM_SHARED((NPAD, D), _f32),
        pltpu.SemaphoreType.DMA,
        pltpu.SemaphoreType.DMA,
    ],
    mesh=_mesh,
)
def _sc_segmean(table, src2, dst2, zrows, parts,
                sidx2, didx2, rows2,
                acc, gsem, ssem):
    cid = lax.axis_index("c")
    sid = lax.axis_index("s")

    pltpu.sync_copy(zrows, acc.at[pl.ds(sid * RPS, RPS), :])

    plsc.subcore_barrier()

    def _fire_gather(i, b):
        pltpu.async_copy(table.at[sidx2.at[i]], rows2.at[b], gsem)

    def _wait_gather(i, b):
        pltpu.make_async_copy(table.at[sidx2.at[i]], rows2.at[b], gsem).wait()

    def _fire_scat(i, b):
        pltpu.async_copy(rows2.at[b], acc.at[didx2.at[i]], ssem, add=True)

    def _wait_scat(i, b):
        pltpu.make_async_copy(rows2.at[b], acc.at[didx2.at[i]], ssem).wait()

    nb = jnp.where(cid == 0, SEG_B0, NBT - SEG_B0)
    b0 = jnp.where(cid == 0, sid * SEG_B0,
                   16 * SEG_B0 + sid * (NBT - SEG_B0))

    def _batch(bi, carry):
        ch0 = pl.multiple_of((b0 + bi) * IB, 8)
        pltpu.sync_copy(src2.at[pl.ds(ch0, IB)], sidx2)
        pltpu.sync_copy(dst2.at[pl.ds(ch0, IB)], didx2)
        for b in range(4):
            _fire_gather(b, b)

        def _inner(o, carry2):
            for b in range(4):
                i = o * 4 + b
                _wait_gather(i, b)
                _fire_scat(i, b)
            for b in range(4):
                i = o * 4 + b
                _wait_scat(i, b)
                _fire_gather(i + 4, b)
            return carry2
        lax.fori_loop(0, IB // 4 - 1, _inner, 0)

        for b in range(4):
            i = IB - 4 + b
            _wait_gather(i, b)
            _fire_scat(i, b)
        for b in range(4):
            _wait_scat(IB - 4 + b, b)
        return carry
    lax.fori_loop(0, nb, _batch, 0)

    plsc.subcore_barrier()

    pltpu.sync_copy(acc.at[pl.ds(sid * RPS, RPS), :],
                    parts.at[cid, pl.ds(sid * RPS, RPS), :])


@functools.partial(
    pl.kernel,
    out_type=[
        jax.ShapeDtypeStruct((2, NPAD, D), _f32),
    ],
    scratch_types=[
        pltpu.VMEM((NCH, CH), jnp.int32),
        pltpu.VMEM((CH, D), _f32),
        pltpu.VMEM_SHARED((NPAD, D), _f32),
        pltpu.SemaphoreType.DMA,
    ],
    mesh=_mesh,
)
def _sc_cnt(dst3, zrows, cnt_parts, didx2, onesb, cacc, ssem):
    cid = lax.axis_index("c")
    sid = lax.axis_index("s")
    wid = cid * 16 + sid

    pltpu.sync_copy(dst3.at[wid], didx2)
    pltpu.sync_copy(zrows, cacc.at[pl.ds(sid * RPS, RPS), :])

    one16 = jnp.full((16,), 1.0, _f32)

    def _fill(r, carry):
        for j in range(D // 16):
            onesb[r, pl.ds(j * 16, 16)] = one16
        return carry
    lax.fori_loop(0, CH, _fill, 0)
    plsc.subcore_barrier()

    def _fire(i):
        pltpu.async_copy(onesb, cacc.at[didx2.at[i]], ssem, add=True)

    def _drain(i):
        pltpu.make_async_copy(onesb, cacc.at[didx2.at[i]], ssem).wait()

    for b in range(8):
        _fire(b)

    def _outer(o, carry):
        for b in range(8):
            i = o * 8 + b
            _drain(i)
            _fire(i + 8)
        return carry
    lax.fori_loop(0, NCH // 8 - 1, _outer, 0)

    for b in range(8):
        _drain(NCH - 8 + b)

    plsc.subcore_barrier()
    pltpu.sync_copy(cacc.at[pl.ds(sid * RPS, RPS), :],
                    cnt_parts.at[cid, pl.ds(sid * RPS, RPS), :])


CE = 64
RE = 4
EB = 16
NBT_E = 20
EDGE_B0 = 15


@functools.partial(
    pl.kernel,
    out_type=[jax.ShapeDtypeStruct((EPAD * 16,), _f32)],
    scratch_types=[
        pltpu.VMEM((EB, CE), jnp.int32),
        pltpu.VMEM((EB, CE), jnp.int32),
        pltpu.VMEM((RE, CE, D), _f32),
        pltpu.VMEM((RE, CE, D), _f32),
        pltpu.VMEM((D,), _f32),
        pltpu.VMEM((CE * 16,), _f32),
        pltpu.SemaphoreType.DMA,
    ],
    mesh=_mesh,
)
def _sc_edge(A, B, src2, dst2, wp2, tsum,
             sidx2, didx2, rA, rB, wvec, outb, gsem):
    cid = lax.axis_index("c")
    sid = lax.axis_index("s")

    pltpu.sync_copy(wp2, wvec)
    wl = [wvec[pl.ds(j * 16, 16)] for j in range(D // 16)]

    def _fire(i, k):
        pltpu.async_copy(A.at[sidx2.at[i]], rA.at[k], gsem)
        pltpu.async_copy(B.at[didx2.at[i]], rB.at[k], gsem)

    def _wait(i, k):
        pltpu.make_async_copy(A.at[sidx2.at[i]], rA.at[k], gsem).wait()
        pltpu.make_async_copy(B.at[didx2.at[i]], rB.at[k], gsem).wait()

    nb = jnp.where(cid == 0, EDGE_B0, NBT_E - EDGE_B0)
    b0 = jnp.where(cid == 0, sid * EDGE_B0,
                   16 * EDGE_B0 + sid * (NBT_E - EDGE_B0))

    def _batch(bi, carry):
        ch0 = pl.multiple_of((b0 + bi) * EB, 8)
        pltpu.sync_copy(src2.at[pl.ds(ch0, EB)], sidx2)
        pltpu.sync_copy(dst2.at[pl.ds(ch0, EB)], didx2)

        def _compute(i, k):
            def _edge(e, carry2):
                t = jnp.zeros((16,), _f32)
                for j in range(D // 16):
                    va = rA[k, e, pl.ds(j * 16, 16)]
                    vb = rB[k, e, pl.ds(j * 16, 16)]
                    t = t + jnp.maximum(va + vb, 0.0) * wl[j]
                outb[pl.ds(e * 16, 16)] = t
                return carry2
            lax.fori_loop(0, CE, _edge, 0)
            pltpu.sync_copy(outb,
                            tsum.at[pl.ds((ch0 + i) * CE * 16, CE * 16)])

        for k in range(RE):
            _fire(k, k)

        def _outer(o, carry2):
            for k in range(RE):
                i = o * RE + k
                _wait(i, k)
                _compute(i, k)
                _fire(i + RE, k)
            return carry2
        lax.fori_loop(0, EB // RE - 1, _outer, 0)

        for k in range(RE):
            i = EB - RE + k
            _wait(i, k)
            _compute(i, k)
        return carry
    lax.fori_loop(0, nb, _batch, 0)


def _mmT(a, w):
    return lax.dot_general(a, w, (((1,), (1,)), ((), ())),
                           preferred_element_type=_f32)


def _tc_inv_body(cnt_ref, inv_ref):
    cnt = cnt_ref[0, :, 0] + cnt_ref[1, :, 0]
    inv_ref[...] = (1.0 / jnp.maximum(cnt, 1.0))[:, None]


def _tc_pre_body(x_ref, wl_ref, wr_ref, g_ref, r_ref):
    x = x_ref[...]
    g_ref[...] = _mmT(x, wl_ref[...])
    r_ref[...] = _mmT(x, wr_ref[...])


def _tc_mid_body(parts_ref, inv_ref, r_ref, bl_ref, wln_ref, wrn_ref,
                 g_ref, rn_ref):
    h = jnp.maximum((parts_ref[0] + parts_ref[1]) * inv_ref[...]
                    + bl_ref[...] + r_ref[...], 0.0)
    g_ref[...] = _mmT(h, wln_ref[...])
    rn_ref[...] = _mmT(h, wrn_ref[...])


def _tc_fin_body(parts_ref, inv_ref, r_ref, bl_ref, wc_ref, bc_ref,
                 wps_ref, wpd_ref, bp1_ref, ne_ref, sup_ref, a_ref, b_ref):
    ne = (parts_ref[0] + parts_ref[1]) * inv_ref[...] + bl_ref[...] + r_ref[...]
    ne_ref[...] = ne
    sup_ref[...] = _mmT(ne, wc_ref[...]) + bc_ref[...]
    a_ref[...] = _mmT(ne, wps_ref[...]) + bp1_ref[...]
    b_ref[...] = _mmT(ne, wpd_ref[...])


def _tc_edge_fin_body(t_ref, m_ref, out_ref):
    out_ref[...] = lax.dot_general(t_ref[...], m_ref[...],
                                   (((1,), (0,)), ((), ())),
                                   preferred_element_type=_f32)


def _tc(body, out_shape, *args):
    return pl.pallas_call(body, out_shape=out_shape)(*args)


def kernel(x, edge_index, Wl1, bl1, Wr1, Wl2, bl2, Wr2, Wl3, bl3, Wr3,
           Wp1, bp1, Wp2, bp2, Wc, bc):
    src = edge_index[0]
    dst = edge_index[1]
    src_p = jnp.concatenate([src, jnp.zeros((EPAD - E,), jnp.int32)])
    dst_p = jnp.concatenate([dst, jnp.full((EPAD - E,), N + 100, jnp.int32)])
    src2g = src_p.reshape(EPAD // CG, CG)
    dst2g = dst_p.reshape(EPAD // CG, CG)
    src2e = src_p.reshape(EPAD // CE, CE)
    dst2e = dst_p.reshape(EPAD // CE, CE)
    dst3c = dst_p.reshape(NW, NCH, CH)
    x_p = jnp.pad(x, ((0, NPAD - N), (0, 0)))
    zrows = jnp.zeros((RPS, D), _f32)
    Wc_p = jnp.pad(Wc, ((0, D - Wc.shape[0]), (0, 0)))
    bc_p = jnp.pad(bc, (0, D - bc.shape[0])).reshape(1, D)
    Wp1s = Wp1[:, :D]
    Wp1d = Wp1[:, D:]

    sds = jax.ShapeDtypeStruct
    g1, r1 = _tc(_tc_pre_body,
                 [sds((NPAD, D), _f32), sds((NPAD, D), _f32)],
                 x_p, Wl1, Wr1)
    (parts1,) = _sc_segmean(g1, src2g, dst2g, zrows)
    (cnt_parts,) = _sc_cnt(dst3c, zrows)
    inv = _tc(_tc_inv_body, sds((NPAD, 1), _f32), cnt_parts)

    g2, r2 = _tc(_tc_mid_body,
                 [sds((NPAD, D), _f32), sds((NPAD, D), _f32)],
                 parts1, inv, r1, bl1.reshape(1, D), Wl2, Wr2)
    (parts2,) = _sc_segmean(g2, src2g, dst2g, zrows)

    g3, r3 = _tc(_tc_mid_body,
                 [sds((NPAD, D), _f32), sds((NPAD, D), _f32)],
                 parts2, inv, r2, bl2.reshape(1, D), Wl3, Wr3)
    (parts3,) = _sc_segmean(g3, src2g, dst2g, zrows)

    ne, sup, Ab, Bb = _tc(
        _tc_fin_body,
        [sds((NPAD, D), _f32), sds((NPAD, D), _f32),
         sds((NPAD, D), _f32), sds((NPAD, D), _f32)],
        parts3, inv, r3, bl3.reshape(1, D), Wc_p, bc_p,
        Wp1s, Wp1d, bp1.reshape(1, D))

    (tsum,) = _sc_edge(Ab, Bb, src2e, dst2e, Wp2.reshape(D))
    msel = jnp.repeat(jnp.eye(8, dtype=_f32), 16, axis=0)
    flow = _tc(_tc_edge_fin_body, sds((EPAD // 8, 8), _f32),
               tsum.reshape(EPAD // 8, D), msel)

    node_embeddings = ne[:N]
    carbon_flows = flow.reshape(EPAD, 1)[:E] + bp2
    supplier_classes = sup[:N, :4]
    return (node_embeddings, carbon_flows, supplier_classes)
